# baseline (speedup 1.0000x reference)
import functools

import jax
import jax.numpy as jnp
from jax.experimental import pallas as pl
from jax.experimental.pallas import tpu as pltpu

_S3 = 0.5773502691896258


def _tp_kernel(x_ref, y_ref, o_ref):
    x = x_ref[...]
    y = y_ref[...]
    te = x.shape[0]

    m = jax.lax.broadcasted_iota(jnp.int32, (te, 256), 1) % 4

    x1r = pltpu.roll(x, 1, axis=1)
    y1l = pltpu.roll(y, 255, axis=1)
    y2l = pltpu.roll(y, 254, axis=1)
    y2r = pltpu.roll(y, 2, axis=1)
    y3r = pltpu.roll(y, 3, axis=1)

    xu = jnp.where((m == 1) | (m == 2), x1r, x)
    yu = jnp.where(m == 1, y1l, y)
    yu = jnp.where(m == 2, y2r, yu)
    yu = jnp.where(m == 3, y3r, yu)
    cu = jnp.where(m == 0, 1.0, _S3).astype(x.dtype)
    u = xu * yu * cu

    xv = jnp.where(m == 1, x1r, x)
    yv = jnp.where(m == 0, y1l, y2l)
    yv = jnp.where(m == 2, y2r, yv)
    q = x * y
    qs = q + pltpu.roll(q, 1, axis=1) + pltpu.roll(q, 2, axis=1)
    v = jnp.where(m == 3, qs, xv * yv) * _S3

    ru = jnp.repeat(u, 2, axis=1)
    rv = jnp.repeat(v, 2, axis=1)
    l2 = jax.lax.broadcasted_iota(jnp.int32, ru.shape, 1) % 2
    o_ref[...] = jnp.where(l2 == 0, ru, rv)


@jax.jit
def kernel(x, y):
    E, C, D = x.shape
    xv = x.reshape(E, C * D)
    yv = y.reshape(E, C * D)
    te = 128
    out = pl.pallas_call(
        _tp_kernel,
        grid=(E // te,),
        in_specs=[
            pl.BlockSpec((te, C * D), lambda i: (i, 0)),
            pl.BlockSpec((te, C * D), lambda i: (i, 0)),
        ],
        out_specs=pl.BlockSpec((te, 2 * C * D), lambda i: (i, 0)),
        out_shape=jax.ShapeDtypeStruct((E, 2 * C * D), x.dtype),
        compiler_params=pltpu.CompilerParams(
            dimension_semantics=("arbitrary",),
        ),
    )(xv, yv)
    return out.reshape(E, C, 2 * D)


# speedup vs baseline: 17.6595x; 17.6595x over previous
import numpy as np

import jax
import jax.numpy as jnp
from jax.experimental import pallas as pl
from jax.experimental.pallas import tpu as pltpu

_S3 = 0.5773502691896258


def _perm_matrix():
    p = np.zeros((512, 512), dtype=np.float32)
    for c in range(64):
        for j in range(8):
            src = 4 * c + j if j < 4 else 256 + 4 * c + (j - 4)
            p[src, 8 * c + j] = 1.0
    return p


_P = _perm_matrix()


def _tp_kernel(x_ref, y_ref, p_ref, o_ref):
    x = x_ref[...]
    y = y_ref[...]

    m = jax.lax.broadcasted_iota(jnp.int32, x.shape, 1) % 4

    x1 = pltpu.roll(x, 1, axis=1)
    x2 = pltpu.roll(x, 2, axis=1)
    x3 = pltpu.roll(x, 3, axis=1)
    y1 = pltpu.roll(y, 1, axis=1)
    y2 = pltpu.roll(y, 2, axis=1)

    bx0 = jnp.where(m == 1, x1, x)
    bx0 = jnp.where(m == 2, x2, bx0)
    bx0 = jnp.where(m == 3, x3, bx0)
    ca = jnp.where(m == 0, 1.0, _S3).astype(x.dtype)
    a = bx0 * y * ca

    by0 = jnp.where(m == 1, y1, y)
    by0 = jnp.where(m == 2, y2, by0)
    xs = pltpu.roll(x, 255, axis=1)
    q = x * y
    w = q + pltpu.roll(q, 255, axis=1) + pltpu.roll(q, 254, axis=1)
    b = jnp.where(m == 3, pltpu.roll(w, 2, axis=1), xs * by0) * _S3

    lhs = jnp.concatenate([a, b], axis=1)
    o_ref[...] = jnp.dot(lhs, p_ref[...], preferred_element_type=jnp.float32)


@jax.jit
def kernel(x, y):
    E, C, D = x.shape
    xv = x.reshape(E, C * D)
    yv = y.reshape(E, C * D)
    te = 1000
    out = pl.pallas_call(
        _tp_kernel,
        grid=(E // te,),
        in_specs=[
            pl.BlockSpec((te, C * D), lambda i: (i, 0)),
            pl.BlockSpec((te, C * D), lambda i: (i, 0)),
            pl.BlockSpec((2 * C * D, 2 * C * D), lambda i: (0, 0)),
        ],
        out_specs=pl.BlockSpec((te, 2 * C * D), lambda i: (i, 0)),
        out_shape=jax.ShapeDtypeStruct((E, 2 * C * D), x.dtype),
        compiler_params=pltpu.CompilerParams(
            dimension_semantics=("arbitrary",),
        ),
    )(xv, yv, jnp.asarray(_P))
    return out.reshape(E, C, 2 * D)


# speedup vs baseline: 59.1110x; 3.3473x over previous
import jax
import jax.numpy as jnp
from jax.experimental import pallas as pl
from jax.experimental.pallas import tpu as pltpu

_S3 = 0.5773502691896258


def _tp_kernel(x_ref, y_ref, o_ref):
    x0 = x_ref[:, :, 0, :]
    x1 = x_ref[:, :, 1, :]
    x2 = x_ref[:, :, 2, :]
    x3 = x_ref[:, :, 3, :]
    y0 = y_ref[:, :, 0, :]
    y1 = y_ref[:, :, 1, :]
    y2 = y_ref[:, :, 2, :]
    y3 = y_ref[:, :, 3, :]

    o_ref[:, :, 0, :] = x0 * y0
    sx0 = _S3 * x0
    o_ref[:, :, 1, :] = sx0 * y1
    o_ref[:, :, 2, :] = sx0 * y2
    o_ref[:, :, 3, :] = sx0 * y3
    sy0 = _S3 * y0
    o_ref[:, :, 4, :] = x1 * sy0
    o_ref[:, :, 5, :] = x2 * sy0
    o_ref[:, :, 6, :] = x3 * sy0
    o_ref[:, :, 7, :] = _S3 * (x1 * y1 + x2 * y2 + x3 * y3)


@jax.jit
def kernel(x, y):
    E, C, D = x.shape
    ET = E // 128

    def to_slabs(a):
        return a.reshape(ET, 128, C, D).transpose(2, 0, 3, 1)

    xv = to_slabs(x)
    yv = to_slabs(y)

    tb = next(d for d in (10, 5, 2, 1) if ET % d == 0)
    out = pl.pallas_call(
        _tp_kernel,
        grid=(ET // tb,),
        in_specs=[
            pl.BlockSpec((C, tb, D, 128), lambda i: (0, i, 0, 0)),
            pl.BlockSpec((C, tb, D, 128), lambda i: (0, i, 0, 0)),
        ],
        out_specs=pl.BlockSpec((C, tb, 2 * D, 128), lambda i: (0, i, 0, 0)),
        out_shape=jax.ShapeDtypeStruct((C, ET, 2 * D, 128), x.dtype),
        compiler_params=pltpu.CompilerParams(
            dimension_semantics=("arbitrary",),
        ),
    )(xv, yv)

    return (out.transpose(1, 3, 0, 2)
               .reshape(E, C, 2 * D))


# speedup vs baseline: 91.0223x; 1.5399x over previous
import jax
import jax.numpy as jnp
from jax.experimental import pallas as pl
from jax.experimental.pallas import tpu as pltpu

_S3 = 0.5773502691896258


def _tp_kernel(x_ref, y_ref, o_ref):
    x = x_ref[...]
    y = y_ref[...]
    n, t = x.shape[0], x.shape[1]

    ci = jax.lax.broadcasted_iota(jnp.int32, (1, 1, 4, 1), 2)
    cvec = jnp.where(ci == 0, 1.0, _S3).astype(x.dtype)
    o_ref[:, :, 0:4, :] = x[:, :, 0:1, :] * y * cvec
    xs = x[:, :, 1:4, :]
    o_ref[:, :, 4:7, :] = (_S3 * xs) * y[:, :, 0:1, :]
    q = xs * y[:, :, 1:4, :]
    d = q[:, :, 0, :] + q[:, :, 1, :] + q[:, :, 2, :]
    o_ref[:, :, 7, :] = _S3 * d


@jax.jit
def kernel(x, y):
    E, C, D = x.shape
    ET = E // 128

    def to_slabs(a):
        return a.reshape(ET, 128, C, D).transpose(2, 0, 3, 1)

    xv = to_slabs(x)
    yv = to_slabs(y)

    tb = next(d for d in (10, 5, 2, 1) if ET % d == 0)
    out = pl.pallas_call(
        _tp_kernel,
        grid=(ET // tb,),
        in_specs=[
            pl.BlockSpec((C, tb, D, 128), lambda i: (0, i, 0, 0)),
            pl.BlockSpec((C, tb, D, 128), lambda i: (0, i, 0, 0)),
        ],
        out_specs=pl.BlockSpec((C, tb, 2 * D, 128), lambda i: (0, i, 0, 0)),
        out_shape=jax.ShapeDtypeStruct((C, ET, 2 * D, 128), x.dtype),
        compiler_params=pltpu.CompilerParams(
            dimension_semantics=("arbitrary",),
        ),
    )(xv, yv)

    return (out.transpose(1, 3, 0, 2)
               .reshape(E, C, 2 * D))


# speedup vs baseline: 106.7399x; 1.1727x over previous
import jax
import jax.numpy as jnp
from jax.experimental import pallas as pl
from jax.experimental.pallas import tpu as pltpu

_S3 = 0.5773502691896258


def _tp_kernel(x_ref, y_ref, o_ref):
    x = x_ref[...]
    y = y_ref[...]
    n, t = x.shape[0], x.shape[1]

    ci = jax.lax.broadcasted_iota(jnp.int32, (1, 1, 4, 1), 2)
    cvec = jnp.where(ci == 0, 1.0, _S3).astype(x.dtype)
    o_ref[:, :, 0:4, :] = x[:, :, 0:1, :] * y * cvec
    xs = x[:, :, 1:4, :]
    o_ref[:, :, 4:7, :] = (_S3 * xs) * y[:, :, 0:1, :]
    q = xs * y[:, :, 1:4, :]
    d = q[:, :, 0, :] + q[:, :, 1, :] + q[:, :, 2, :]
    o_ref[:, :, 7, :] = _S3 * d


@jax.jit
def kernel(x, y):
    E, C, D = x.shape
    ET = E // 128

    def to_slabs(a):
        return a.reshape(ET, 128, C, D).transpose(2, 0, 3, 1)

    xv = to_slabs(x)
    yv = to_slabs(y)

    tb = next(d for d in (25, 10, 5, 2, 1) if ET % d == 0)
    out = pl.pallas_call(
        _tp_kernel,
        grid=(ET // tb,),
        in_specs=[
            pl.BlockSpec((C, tb, D, 128), lambda i: (0, i, 0, 0)),
            pl.BlockSpec((C, tb, D, 128), lambda i: (0, i, 0, 0)),
        ],
        out_specs=pl.BlockSpec((C, tb, 2 * D, 128), lambda i: (0, i, 0, 0)),
        out_shape=jax.ShapeDtypeStruct((C, ET, 2 * D, 128), x.dtype),
        compiler_params=pltpu.CompilerParams(
            dimension_semantics=("arbitrary",),
        ),
    )(xv, yv)

    return (out.transpose(1, 3, 0, 2)
               .reshape(E, C, 2 * D))
